# Initial kernel scaffold; baseline (speedup 1.0000x reference)
import jax
import jax.numpy as jnp
from jax.experimental import pallas as pl


def kernel(inp1, inp2):
    raise NotImplementedError("write your pallas kernel here")


# speedup vs baseline: 1.1540x; 1.1540x over previous
import functools

import jax
import jax.numpy as jnp
from jax import lax
from jax.experimental import pallas as pl
from jax.experimental.pallas import tpu as pltpu
from jax.experimental.pallas import tpu_sc as plsc

N = 16384
C = 128
TBL = 128


def _table_body(inp1_ref, tbl_ref):
    x = inp1_ref[...]
    xt = jnp.transpose(x, (1, 0))
    tbl_ref[...] = xt * xt


def _build_table(inp1):
    return pl.pallas_call(
        _table_body,
        grid=(1,),
        in_specs=[pl.BlockSpec((C, TBL), lambda j: (0, 0))],
        out_specs=pl.BlockSpec((TBL, C), lambda j: (0, 0)),
        out_shape=jax.ShapeDtypeStruct((TBL, C), jnp.float32),
    )(inp1)


@functools.cache
def _make_sc_gather():
    info = plsc.get_sparse_core_info()
    nc, ns, nl = info.num_cores, info.num_subcores, info.num_lanes
    nw = nc * ns
    b_per_w = N // nw
    mesh = plsc.VectorSubcoreMesh(core_axis_name="c", subcore_axis_name="s")

    @functools.partial(
        pl.kernel,
        mesh=mesh,
        out_type=jax.ShapeDtypeStruct((N, C), jnp.float32),
        scratch_types=[
            pltpu.VMEM((b_per_w,), jnp.int32),
            pltpu.VMEM((b_per_w, C), jnp.float32),
            pltpu.SemaphoreType.DMA,
        ],
    )
    def gather_k(table_hbm, idx_hbm, out_hbm, idx_v, rows_v, sem):
        wid = lax.axis_index("s") * nc + lax.axis_index("c")
        base = wid * b_per_w
        pltpu.sync_copy(idx_hbm.at[pl.ds(base, b_per_w)], idx_v)
        for i in range(b_per_w // nl):
            v = idx_v[pl.ds(i * nl, nl)]
            idx_v[pl.ds(i * nl, nl)] = jnp.minimum(jnp.maximum(v, 0), 63)
        pltpu.async_copy(table_hbm.at[idx_v], rows_v, sem).wait()
        pltpu.sync_copy(rows_v, out_hbm.at[pl.ds(base, b_per_w)])

    return gather_k


_BLK = 2048


def _mul_body(inp1_ref, g_ref, o_ref):
    o_ref[...] = jnp.transpose(inp1_ref[...], (1, 0)) * g_ref[...]


def _mul(inp1, g):
    return pl.pallas_call(
        _mul_body,
        grid=(N // _BLK,),
        in_specs=[
            pl.BlockSpec((C, _BLK), lambda j: (0, j)),
            pl.BlockSpec((_BLK, C), lambda j: (j, 0)),
        ],
        out_specs=pl.BlockSpec((_BLK, C), lambda j: (j, 0)),
        out_shape=jax.ShapeDtypeStruct((N, C), jnp.float32),
    )(inp1, g)


def kernel(inp1, inp2):
    idx = inp2.reshape(N).astype(jnp.int32)
    table = _build_table(inp1)
    g = _make_sc_gather()(table, idx)
    out = _mul(inp1, g)
    return (out,)
